# Initial kernel scaffold; baseline (speedup 1.0000x reference)
import jax
import jax.numpy as jnp
from jax.experimental import pallas as pl


def kernel(x, weight):
    raise NotImplementedError("write your pallas kernel here")


# speedup vs baseline: 4.6585x; 4.6585x over previous
import functools

import jax
import jax.numpy as jnp
from jax import lax
from jax.experimental import pallas as pl
from jax.experimental.pallas import tpu as pltpu
from jax.experimental.pallas import tpu_sc as plsc

_INFO = plsc.get_sparse_core_info()
_NC = _INFO.num_cores
_NS = _INFO.num_subcores
_NW = _NC * _NS


@functools.partial(jax.jit, static_argnames=("n_per_w", "chunk"))
def _sc_gather(idx, weight, *, n_per_w, chunk):
    n_total, = idx.shape
    _, d = weight.shape
    n_chunks = n_per_w // chunk
    mesh = plsc.VectorSubcoreMesh(core_axis_name="c", subcore_axis_name="s")

    @functools.partial(
        pl.kernel,
        mesh=mesh,
        out_type=jax.ShapeDtypeStruct((n_total, d), jnp.float32),
        scratch_types=[
            pltpu.VMEM((chunk,), jnp.int32),
            pltpu.VMEM((chunk,), jnp.int32),
            pltpu.VMEM((chunk, d), jnp.float32),
            pltpu.VMEM((chunk, d), jnp.float32),
            pltpu.SemaphoreType.DMA,
            pltpu.SemaphoreType.DMA,
        ],
        compiler_params=pltpu.CompilerParams(use_tc_tiling_on_sc=False),
    )
    def k(idx_hbm, table_hbm, out_hbm, idx0, idx1, rows0, rows1, sem0, sem1):
        wid = lax.axis_index("s") * _NC + lax.axis_index("c")
        w_base = wid * n_per_w
        idx_b = (idx0, idx1)
        rows_b = (rows0, rows1)
        sems = (sem0, sem1)

        def start(c, slot):
            base = w_base + c * chunk
            pltpu.sync_copy(idx_hbm.at[pl.ds(base, chunk)], idx_b[slot])
            return pltpu.async_copy(
                table_hbm.at[idx_b[slot]], rows_b[slot], sems[slot]
            )

        def drain(c, slot):
            base = w_base + c * chunk
            pltpu.make_async_copy(
                table_hbm.at[idx_b[slot]], rows_b[slot], sems[slot]
            ).wait()
            pltpu.sync_copy(rows_b[slot], out_hbm.at[pl.ds(base, chunk)])

        start(0, 0)
        for c in range(n_chunks):
            slot = c % 2
            if c + 1 < n_chunks:
                start(c + 1, 1 - slot)
            drain(c, slot)

    return k(idx, weight)


def kernel(x, weight):
    b, h = x.shape
    v, d = weight.shape
    n = b * h
    idx = x.reshape(n).astype(jnp.int32)
    n_per_w = n // _NW
    chunk = 800
    out = _sc_gather(idx, weight, n_per_w=n_per_w, chunk=chunk)
    return out.reshape(b, h, d)


# speedup vs baseline: 4.6735x; 1.0032x over previous
import functools

import jax
import jax.numpy as jnp
from jax import lax
from jax.experimental import pallas as pl
from jax.experimental.pallas import tpu as pltpu
from jax.experimental.pallas import tpu_sc as plsc

_INFO = plsc.get_sparse_core_info()
_NC = _INFO.num_cores
_NS = _INFO.num_subcores
_NW = _NC * _NS


@functools.partial(jax.jit, static_argnames=("n_per_w", "chunk", "depth"))
def _sc_gather(idx, weight, *, n_per_w, chunk, depth):
    n_total, = idx.shape
    _, d = weight.shape
    n_chunks = n_per_w // chunk
    mesh = plsc.VectorSubcoreMesh(core_axis_name="c", subcore_axis_name="s")

    @functools.partial(
        pl.kernel,
        mesh=mesh,
        out_type=jax.ShapeDtypeStruct((n_total, d), jnp.float32),
        scratch_types=[
            pltpu.VMEM((n_per_w,), jnp.int32),
            *[pltpu.VMEM((chunk, d), jnp.float32) for _ in range(depth)],
            *[pltpu.SemaphoreType.DMA for _ in range(depth)],
        ],
        compiler_params=pltpu.CompilerParams(use_tc_tiling_on_sc=False),
    )
    def k(idx_hbm, table_hbm, out_hbm, idx_v, *bufs):
        rows_b = bufs[:depth]
        sems = bufs[depth:]
        wid = lax.axis_index("s") * _NC + lax.axis_index("c")
        w_base = wid * n_per_w

        pltpu.sync_copy(idx_hbm.at[pl.ds(w_base, n_per_w)], idx_v)

        def start(c, s):
            pltpu.async_copy(
                table_hbm.at[idx_v.at[pl.ds(c * chunk, chunk)]],
                rows_b[s], sems[s],
            )

        def wait(s):
            pltpu.make_async_copy(
                table_hbm.at[idx_v.at[pl.ds(0, chunk)]], rows_b[s], sems[s]
            ).wait()

        for s in range(min(depth, n_chunks)):
            start(s, s)
        for c in range(n_chunks):
            s = c % depth
            wait(s)
            pltpu.sync_copy(
                rows_b[s], out_hbm.at[pl.ds(w_base + c * chunk, chunk)]
            )
            if c + depth < n_chunks:
                start(c + depth, s)

    return k(idx, weight)


def kernel(x, weight):
    b, h = x.shape
    v, d = weight.shape
    n = b * h
    idx = x.reshape(n).astype(jnp.int32)
    n_per_w = n // _NW
    out = _sc_gather(idx, weight, n_per_w=n_per_w, chunk=400, depth=4)
    return out.reshape(b, h, d)
